# baseline (speedup 1.0000x reference)
import functools

import jax
import jax.numpy as jnp
from jax import lax
from jax.experimental import pallas as pl
from jax.experimental.pallas import tpu as pltpu
from jax.experimental.pallas import tpu_sc as plsc

_SC_INFO = plsc.get_sparse_core_info()
_NC = _SC_INFO.num_cores
_NS = _SC_INFO.num_subcores
_NW = _NC * _NS
_L = _SC_INFO.num_lanes

_F = 4
_S = 8
_CH = 1024
_KB = 5
_SENT = 0x7FFFFFFF
_WIDE = 128


@functools.partial(jax.jit, static_argnames=("batch", "nrows", "dim"))
def _sc_scan_gather(latents, latent_id, batch, nrows, dim):
    table3 = latents.T.reshape(_F, _S, nrows)
    n_full = nrows // _CH
    tail_c0 = n_full * _CH
    tail_len = nrows - tail_c0
    tail_slab = (tail_len // 128) * 128
    tail_owner = n_full % _NW
    mesh = plsc.VectorSubcoreMesh(core_axis_name="c", subcore_axis_name="s")

    @functools.partial(
        pl.kernel,
        mesh=mesh,
        out_type=jax.ShapeDtypeStruct((batch, _WIDE), latents.dtype),
        scratch_types=[
            pltpu.VMEM((batch + _L,), jnp.int32),
            pltpu.VMEM((batch + _L,), jnp.int32),
            pltpu.VMEM((_CH,), jnp.int32),
            pltpu.VMEM((2, _F, _CH // 128, _S, 128), latents.dtype),
            pltpu.VMEM((2 * 64, _WIDE), latents.dtype),
            pltpu.VMEM((2, 64), jnp.int32),
            pltpu.VMEM((_L, dim), latents.dtype),
            pltpu.SemaphoreType.DMA,
            pltpu.SemaphoreType.DMA,
            pltpu.SemaphoreType.DMA,
        ],
        compiler_params=pltpu.CompilerParams(needs_layout_passes=False),
    )
    def body(t3, lat2d, idx_hbm, wide_out, cand, ccand, idx_p, slab, rowbuf,
             posring, minislab, sem_sl, sem_ix, sem_out):
        wid = lax.axis_index("s") * _NC + lax.axis_index("c")
        iota = lax.iota(jnp.int32, _L)
        nfull_w = jnp.where(wid < tail_owner, n_full // _NW + 1, n_full // _NW)

        def issue_chunk(c0, b):
            for f in range(_F):
                for t in range(_CH // 128):
                    pltpu.async_copy(
                        t3.at[f, :, pl.ds(c0 + t * 128, 128)],
                        slab.at[b, f, t],
                        sem_sl,
                    )

        def wait_chunk():
            for f in range(_F):
                for t in range(_CH // 128):
                    pltpu.make_async_copy(
                        t3.at[f, :, pl.ds(0, 128)], slab.at[0, f, t], sem_sl
                    ).wait()

        issue_chunk(wid * _CH, 0)
        issue_chunk((wid + _NW) * _CH, 1)

        def scan_piece(p, cnt):
            pltpu.async_copy(
                idx_hbm.at[pl.ds(p * _CH, _CH)], idx_p, sem_ix
            ).wait()

            def scan_group(g, cnt):
                v = idx_p[pl.ds(g * _L, _L)]
                mine = lax.bitwise_and(
                    lax.shift_right_logical(v, 10), _NW - 1
                ) == wid
                pos = p * _CH + g * _L + iota
                code = lax.bitwise_or(
                    lax.bitwise_or(
                        lax.bitwise_and(v, _CH - 1),
                        lax.shift_left(lax.shift_right_logical(v, 15), 10),
                    ),
                    lax.shift_left(pos, 15),
                )
                mi = mine.astype(jnp.int32)
                rank = plsc.cumsum(mi) - mi
                dest = jnp.where(mine, cnt + rank, batch)
                plsc.store_scatter(cand, [dest], code)
                n = plsc.all_reduce_population_count(mine)[0]
                return cnt + n

            return lax.fori_loop(0, _CH // _L, scan_group, cnt)

        cnt = lax.fori_loop(0, batch // _CH, scan_piece, 0)
        cgroups = lax.div(cnt + _L - 1, _L)

        sentv = jnp.full((_L,), _SENT, jnp.int32)
        for w in range(2):
            for i in range(64 // _L):
                posring[w, pl.ds(i * _L, _L)] = sentv

        def process_chunk(k, b, state, *, tail=False):
            limit = tail_slab if tail else _CH
            c0_chunk = tail_c0 if tail else None
            slot, flushed = state
            bvec = jnp.full((_L,), b, jnp.int32)

            def compact_group(g, m):
                c = cand[pl.ds(g * _L, _L)]
                valid = (g * _L + iota) < cnt
                hit = jnp.logical_and(
                    valid,
                    lax.bitwise_and(
                        lax.shift_right_logical(c, 10), 2 ** _KB - 1
                    ) == k,
                )
                hi = hit.astype(jnp.int32)
                rank = plsc.cumsum(hi) - hi
                dest = jnp.where(hit, m + rank, batch)
                plsc.store_scatter(ccand, [dest], c)
                return m + plsc.all_reduce_population_count(hit)[0]

            m = lax.fori_loop(0, cgroups, compact_group, 0)

            def extract_group(h, state):
                slot, flushed = state
                c = ccand[pl.ds(h * _L, _L)]
                vmask = (h * _L + iota) < m
                nh = plsc.all_reduce_population_count(vmask)[0]
                rel = lax.bitwise_and(c, _CH - 1)
                pos = jnp.where(vmask, lax.shift_right_logical(c, 15), _SENT)
                slotv = lax.bitwise_and(slot + iota, 127)
                plsc.store_scatter(
                    posring,
                    [lax.shift_right_logical(slotv, 6),
                     lax.bitwise_and(slotv, 63)],
                    pos,
                )
                if tail:
                    utmask = jnp.logical_and(vmask, rel >= limit)
                    utbits = utmask.astype(jnp.int32)
                    for l in range(_L):
                        @pl.when(utbits[l] != 0)
                        def _():
                            pltpu.sync_copy(
                                lat2d.at[pl.ds(c0_chunk + rel[l], 1), :],
                                minislab.at[pl.ds(l, 1), :],
                            )
                tvec = lax.shift_right_logical(rel, 7)
                lvec = lax.bitwise_and(rel, 127)
                for c_i in range(dim):
                    val = plsc.load_gather(
                        slab,
                        [bvec,
                         jnp.full((_L,), c_i // _S, jnp.int32),
                         tvec,
                         jnp.full((_L,), c_i % _S, jnp.int32),
                         lvec],
                    )
                    if tail:
                        val_ut = plsc.load_gather(
                            minislab,
                            [iota, jnp.full((_L,), c_i, jnp.int32)],
                        )
                        val = jnp.where(utmask, val_ut, val)
                    plsc.store_scatter(
                        rowbuf,
                        [slotv, jnp.full((_L,), c_i, jnp.int32)],
                        val,
                    )
                slot = slot + nh
                do_flush = (slot - flushed) >= 64

                @pl.when(do_flush)
                def _():
                    w = lax.bitwise_and(
                        lax.shift_right_logical(flushed, 6), 1
                    )
                    pltpu.async_copy(
                        rowbuf.at[pl.ds(w * 64, 64)],
                        wide_out.at[
                            plsc.Indices(posring.at[w], ignored_value=_SENT)
                        ],
                        sem_out,
                    ).wait()
                    for i in range(64 // _L):
                        posring[w, pl.ds(i * _L, _L)] = sentv

                flushed = flushed + 64 * do_flush.astype(jnp.int32)
                return slot, flushed

            hgroups = lax.div(m + _L - 1, _L)
            return lax.fori_loop(0, hgroups, extract_group, (slot, flushed))

        def chunk_step(k, state):
            wait_chunk()
            b = lax.bitwise_and(k, 1)
            state = process_chunk(k, b, state)

            @pl.when(k + 2 < nfull_w)
            def _():
                issue_chunk((wid + (k + 2) * _NW) * _CH, b)

            return state

        state = lax.fori_loop(0, nfull_w, chunk_step, (0, 0))

        def tail_branch(state):
            for f in range(_F):
                for t in range(tail_slab // 128):
                    pltpu.async_copy(
                        t3.at[f, :, pl.ds(tail_c0 + t * 128, 128)],
                        slab.at[0, f, t],
                        sem_sl,
                    )
            for f in range(_F):
                for t in range(tail_slab // 128):
                    pltpu.make_async_copy(
                        t3.at[f, :, pl.ds(0, 128)], slab.at[0, f, t], sem_sl
                    ).wait()
            return process_chunk(n_full // _NW, 0, state, tail=True)

        state = lax.cond(
            wid == tail_owner, tail_branch, lambda s: s, state
        )

        for w in range(2):
            pltpu.async_copy(
                rowbuf.at[pl.ds(w * 64, 64)],
                wide_out.at[plsc.Indices(posring.at[w], ignored_value=_SENT)],
                sem_out,
            ).wait()

    return body(table3, latents, latent_id)


def kernel(latents, latent_id):
    batch = latent_id.shape[0]
    nrows, dim = latents.shape
    wide = _sc_scan_gather(latents, latent_id, batch, nrows, dim)
    return wide[:, :dim]


# speedup vs baseline: 1.2379x; 1.2379x over previous
import functools

import jax
import jax.numpy as jnp
from jax import lax
from jax.experimental import pallas as pl
from jax.experimental.pallas import tpu as pltpu
from jax.experimental.pallas import tpu_sc as plsc

_SC_INFO = plsc.get_sparse_core_info()
_NC = _SC_INFO.num_cores
_NS = _SC_INFO.num_subcores
_NW = _NC * _NS


@functools.partial(jax.jit, static_argnames=("batch", "dim"))
def _sc_gather(latents, latent_id, batch, dim):
    b_per_w = batch // _NW
    mesh = plsc.VectorSubcoreMesh(core_axis_name="c", subcore_axis_name="s")

    @functools.partial(
        pl.kernel,
        mesh=mesh,
        out_type=jax.ShapeDtypeStruct((batch, dim), latents.dtype),
        scratch_types=[
            pltpu.VMEM((b_per_w,), jnp.int32),
            pltpu.VMEM((b_per_w, dim), latents.dtype),
            pltpu.SemaphoreType.DMA,
        ],
    )
    def body(table_hbm, idx_hbm, out_hbm, idx_v, rows_v, sem):
        wid = lax.axis_index("s") * _NC + lax.axis_index("c")
        base = wid * b_per_w
        pltpu.sync_copy(idx_hbm.at[pl.ds(base, b_per_w)], idx_v)

        lanes = _SC_INFO.num_lanes

        def issue(g, carry):
            vec = idx_v[pl.ds(g * lanes, lanes)]
            for k in range(lanes):
                i = vec[k]
                pltpu.async_copy(
                    table_hbm.at[pl.ds(i, 1), :],
                    rows_v.at[pl.ds(g * lanes + k, 1), :],
                    sem,
                )
            return carry

        lax.fori_loop(0, b_per_w // lanes, issue, 0)
        pltpu.make_async_copy(
            table_hbm.at[pl.ds(0, b_per_w), :], rows_v, sem
        ).wait()
        pltpu.sync_copy(rows_v, out_hbm.at[pl.ds(base, b_per_w)])

    return body(latents, latent_id)


def kernel(latents, latent_id):
    batch = latent_id.shape[0]
    dim = latents.shape[1]
    return _sc_gather(latents, latent_id, batch, dim)
